# baseline (speedup 1.0000x reference)
import functools
import math

import jax
import jax.numpy as jnp
from jax import lax
from jax.experimental import pallas as pl
from jax.experimental.pallas import tpu as pltpu

_PAD = 0
_CONF = 0.9
_SEG = 128


def _body(nblk, rpb, V, tok_smem, tok_ref, x_ref, xany_ref, out_ref,
          acc_ref, stage_ref, sem):
    i = pl.program_id(0)

    copies = []
    for r in range(rpb):
        t = tok_smem[r, 0]
        f = (i * rpb + r) * V + t
        c0 = pl.multiple_of((f >> 7) << 7, _SEG)
        copies.append(pltpu.make_async_copy(
            xany_ref.at[pl.ds(c0, _SEG)], stage_ref.at[pl.ds(r * _SEG, _SEG)],
            sem))
    for dma in copies:
        dma.start()

    x = x_ref[...]
    rowsum = jnp.sum(x, axis=1, keepdims=True)
    l0 = x[:, 0:1]

    for dma in copies:
        dma.wait()
    tok = tok_ref[...]
    rows = i * rpb + lax.broadcasted_iota(jnp.int32, (rpb, 1), 0)
    loc = (rows * V + tok) & (_SEG - 1)
    seg = stage_ref[...].reshape(rpb, _SEG)
    seg_col = lax.broadcasted_iota(jnp.int32, (rpb, _SEG), 1)
    le = jnp.sum(jnp.where(seg_col == loc, seg, 0.0), axis=1, keepdims=True)

    d = (1.0 - _CONF) / (V - 2)
    nonpad = (tok != _PAD).astype(jnp.float32)
    contrib = d * (rowsum - l0 - le) + _CONF * le
    tot_part = jnp.sum(contrib * nonpad)
    n_part = jnp.sum(nonpad)

    @pl.when(i == 0)
    def _():
        acc_ref[...] = jnp.zeros_like(acc_ref)

    acc_ref[0:1, 0:1] += jnp.full((1, 1), tot_part)
    acc_ref[1:2, 0:1] += jnp.full((1, 1), n_part)

    @pl.when(i == nblk - 1)
    def _():
        a_const = (V - 2) * d * math.log(d) + _CONF * math.log(_CONF)
        tot = acc_ref[0, 0]
        n = acc_ref[1, 0]
        loss = (n * a_const - tot) / jnp.maximum(n, 1.0)
        out_ref[...] = jnp.full(out_ref.shape, loss)


def kernel(vocab_logits, expected_output_tokens, batch_idx):
    B, S, V = vocab_logits.shape
    R = B * S
    x2 = vocab_logits.reshape(R, V)
    tok2 = expected_output_tokens.reshape(R, 1)
    rpb = 32
    nblk = R // rpb
    out = pl.pallas_call(
        functools.partial(_body, nblk, rpb, V),
        grid=(nblk,),
        in_specs=[
            pl.BlockSpec((rpb, 1), lambda i: (i, 0), memory_space=pltpu.SMEM),
            pl.BlockSpec((rpb, 1), lambda i: (i, 0)),
            pl.BlockSpec((rpb, V), lambda i: (i, 0)),
            pl.BlockSpec(memory_space=pl.ANY),
        ],
        out_specs=pl.BlockSpec((8, 128), lambda i: (0, 0)),
        out_shape=jax.ShapeDtypeStruct((8, 128), jnp.float32),
        scratch_shapes=[
            pltpu.VMEM((8, 128), jnp.float32),
            pltpu.VMEM((rpb * _SEG,), jnp.float32),
            pltpu.SemaphoreType.DMA,
        ],
    )(tok2, tok2, x2, vocab_logits.reshape(R * V))
    return out[0, 0]


# speedup vs baseline: 4.7098x; 4.7098x over previous
import functools
import math

import jax
import jax.numpy as jnp
from jax import lax
from jax.experimental import pallas as pl
from jax.experimental.pallas import tpu as pltpu

_PAD = 0
_CONF = 0.9


def _body(nblk, rpb, V, tok_ref, x_ref, out_ref, acc_ref):
    i = pl.program_id(0)
    x = x_ref[...]
    tok = tok_ref[...]
    col = lax.broadcasted_iota(jnp.int32, x.shape, 1)
    d = (1.0 - _CONF) / (V - 2)
    kw = (_CONF - d) / d
    sw = jnp.sum(jnp.where(col == tok, (1.0 + kw) * x, x), axis=1, keepdims=True)
    l0 = x[:, 0:1]
    nonpad = (tok != _PAD).astype(jnp.float32)
    contrib = d * (sw - l0)
    tot_part = jnp.sum(contrib * nonpad)
    n_part = jnp.sum(nonpad)

    @pl.when(i == 0)
    def _():
        acc_ref[...] = jnp.zeros_like(acc_ref)

    acc_ref[0:1, 0:1] += jnp.full((1, 1), tot_part)
    acc_ref[1:2, 0:1] += jnp.full((1, 1), n_part)

    @pl.when(i == nblk - 1)
    def _():
        a_const = (V - 2) * d * math.log(d) + _CONF * math.log(_CONF)
        tot = acc_ref[0, 0]
        n = acc_ref[1, 0]
        loss = (n * a_const - tot) / jnp.maximum(n, 1.0)
        out_ref[...] = jnp.full(out_ref.shape, loss)


def kernel(vocab_logits, expected_output_tokens, batch_idx):
    B, S, V = vocab_logits.shape
    R = B * S
    x2 = vocab_logits.reshape(R, V)
    tok2 = expected_output_tokens.reshape(R, 1)
    rpb = 64
    nblk = R // rpb
    out = pl.pallas_call(
        functools.partial(_body, nblk, rpb, V),
        grid=(nblk,),
        in_specs=[
            pl.BlockSpec((rpb, 1), lambda i: (i, 0)),
            pl.BlockSpec((rpb, V), lambda i: (i, 0)),
        ],
        out_specs=pl.BlockSpec((8, 128), lambda i: (0, 0)),
        out_shape=jax.ShapeDtypeStruct((8, 128), jnp.float32),
        scratch_shapes=[pltpu.VMEM((8, 128), jnp.float32)],
    )(tok2, x2)
    return out[0, 0]


# speedup vs baseline: 5.0245x; 1.0668x over previous
import functools
import math

import jax
import jax.numpy as jnp
from jax import lax
from jax.experimental import pallas as pl
from jax.experimental.pallas import tpu as pltpu

_PAD = 0
_CONF = 0.9


def _body(nblk, rpb, V, tok_ref, x_ref, out_ref, acc_ref):
    i = pl.program_id(0)
    x = x_ref[...]
    tok = tok_ref[...]
    col = lax.broadcasted_iota(jnp.int32, x.shape, 1)
    d = (1.0 - _CONF) / (V - 2)
    kw = (_CONF - d) / d
    sw = jnp.sum(jnp.where(col == tok, (1.0 + kw) * x, x), axis=1, keepdims=True)
    l0 = x[:, 0:1]
    nonpad = (tok != _PAD).astype(jnp.float32)
    contrib = d * (sw - l0)
    tot_part = jnp.sum(contrib * nonpad)
    n_part = jnp.sum(nonpad)

    @pl.when(i == 0)
    def _():
        acc_ref[...] = jnp.zeros_like(acc_ref)

    acc_ref[0:1, 0:1] += jnp.full((1, 1), tot_part)
    acc_ref[1:2, 0:1] += jnp.full((1, 1), n_part)

    @pl.when(i == nblk - 1)
    def _():
        a_const = (V - 2) * d * math.log(d) + _CONF * math.log(_CONF)
        tot = acc_ref[0, 0]
        n = acc_ref[1, 0]
        loss = (n * a_const - tot) / jnp.maximum(n, 1.0)
        out_ref[...] = jnp.full(out_ref.shape, loss)


def kernel(vocab_logits, expected_output_tokens, batch_idx):
    B, S, V = vocab_logits.shape
    R = B * S
    x2 = vocab_logits.reshape(R, V)
    tok2 = expected_output_tokens.reshape(R, 1)
    rpb = 32
    nblk = R // rpb
    out = pl.pallas_call(
        functools.partial(_body, nblk, rpb, V),
        grid=(nblk,),
        in_specs=[
            pl.BlockSpec((rpb, 1), lambda i: (i, 0)),
            pl.BlockSpec((rpb, V), lambda i: (i, 0)),
        ],
        out_specs=pl.BlockSpec((8, 128), lambda i: (0, 0)),
        out_shape=jax.ShapeDtypeStruct((8, 128), jnp.float32),
        scratch_shapes=[pltpu.VMEM((8, 128), jnp.float32)],
    )(tok2, x2)
    return out[0, 0]


# speedup vs baseline: 5.2198x; 1.0389x over previous
import functools
import math

import jax
import jax.numpy as jnp
from jax import lax
from jax.experimental import pallas as pl
from jax.experimental.pallas import tpu as pltpu

_PAD = 0
_CONF = 0.9


def _body(nblk, rpb, V, tok_ref, x_ref, out_ref, acc_ref):
    i = pl.program_id(0)
    x = x_ref[...]
    tok = tok_ref[...]
    col = lax.broadcasted_iota(jnp.int32, x.shape, 1)
    d = (1.0 - _CONF) / (V - 2)
    kw = (_CONF - d) / d
    sw = jnp.sum(jnp.where(col == tok, (1.0 + kw) * x, x), axis=1, keepdims=True)
    l0 = x[:, 0:1]
    nonpad = (tok != _PAD).astype(jnp.float32)
    contrib = d * (sw - l0)
    tot_part = jnp.sum(contrib * nonpad)
    n_part = jnp.sum(nonpad)

    @pl.when(i == 0)
    def _():
        acc_ref[...] = jnp.zeros_like(acc_ref)

    acc_ref[0:1, 0:1] += jnp.full((1, 1), tot_part)
    acc_ref[1:2, 0:1] += jnp.full((1, 1), n_part)

    @pl.when(i == nblk - 1)
    def _():
        a_const = (V - 2) * d * math.log(d) + _CONF * math.log(_CONF)
        tot = acc_ref[0, 0]
        n = acc_ref[1, 0]
        loss = (n * a_const - tot) / jnp.maximum(n, 1.0)
        out_ref[0, 0] = loss


def kernel(vocab_logits, expected_output_tokens, batch_idx):
    B, S, V = vocab_logits.shape
    R = B * S
    x2 = vocab_logits.reshape(R, V)
    tok2 = expected_output_tokens.reshape(R, 1)
    rpb = 32
    nblk = R // rpb
    out = pl.pallas_call(
        functools.partial(_body, nblk, rpb, V),
        grid=(nblk,),
        in_specs=[
            pl.BlockSpec((rpb, 1), lambda i: (i, 0)),
            pl.BlockSpec((rpb, V), lambda i: (i, 0)),
        ],
        out_specs=pl.BlockSpec(memory_space=pltpu.SMEM),
        out_shape=jax.ShapeDtypeStruct((1, 1), jnp.float32),
        scratch_shapes=[pltpu.VMEM((8, 128), jnp.float32)],
    )(tok2, x2)
    return out.reshape(())
